# baseline (speedup 1.0000x reference)
import functools

import jax
import jax.numpy as jnp
from jax import lax
from jax.experimental import pallas as pl
from jax.experimental.pallas import tpu as pltpu
from jax.experimental.pallas import tpu_sc as plsc

D = 64
SEQ = 200
SCALE = 8.0
LANES = 16
NG = 128 // LANES

_info = plsc.get_sparse_core_info()
_NC = _info.num_cores
_NS = _info.num_subcores
_NW = _NC * _NS


def _emb_kernel(B: int):
    bw = B // _NW
    mesh = plsc.VectorSubcoreMesh(core_axis_name="c", subcore_axis_name="s")

    @functools.partial(
        pl.kernel,
        mesh=mesh,
        out_type=jax.ShapeDtypeStruct((SEQ, D, B), jnp.float32),
        scratch_types=[
            pltpu.VMEM((bw,), jnp.int32),
            pltpu.VMEM((bw,), jnp.int32),
            pltpu.VMEM((bw, 2 * D), jnp.float32),
            pltpu.VMEM((D, bw), jnp.float32),
            pltpu.VMEM((D, bw), jnp.float32),
            pltpu.SemaphoreType.DMA,
            pltpu.SemaphoreType.DMA,
            pltpu.SemaphoreType.DMA,
        ],
        compiler_params=pltpu.CompilerParams(
            use_tc_tiling_on_sc=False, needs_layout_passes=False
        ),
    )
    def k(idx_hbm, table_hbm, out_hbm,
          idx_v, half_v, pairs_v, slab0, slab1, gsem, osem0, osem1):
        wid = lax.axis_index("s") * _NC + lax.axis_index("c")
        b0 = wid * bw
        slab = (slab0, slab1)
        osem = (osem0, osem1)

        def step(s, b):
            pltpu.sync_copy(idx_hbm.at[s, pl.ds(b0, bw)], idx_v)
            for g in range(NG):
                sl = pl.ds(g * LANES, LANES)
                half_v[sl] = lax.shift_right_logical(idx_v[sl], 1)
            pltpu.async_copy(table_hbm.at[half_v], pairs_v, gsem).wait()

            @pl.when(s >= 2)
            def _():
                pltpu.make_async_copy(
                    slab[b], out_hbm.at[s - 2, :, pl.ds(b0, bw)], osem[b]
                ).wait()

            for g in range(NG):
                sl = pl.ds(g * LANES, LANES)
                rowi = jax.lax.iota(jnp.int32, LANES) + (g * LANES)
                coli = (idx_v[sl] & 1) * D

                def dbody(d, _):
                    vals = plsc.load_gather(pairs_v, [rowi, coli + d])
                    slab[b][d, sl] = vals * SCALE
                    return 0

                lax.fori_loop(0, D, dbody, 0)

            pltpu.make_async_copy(
                slab[b], out_hbm.at[s, :, pl.ds(b0, bw)], osem[b]
            ).start()

        def pair_body(p, _):
            step(2 * p, 0)
            step(2 * p + 1, 1)
            return 0

        lax.fori_loop(0, SEQ // 2, pair_body, 0)

        pltpu.make_async_copy(
            slab[0], out_hbm.at[SEQ - 2, :, pl.ds(b0, bw)], osem[0]
        ).wait()
        pltpu.make_async_copy(
            slab[1], out_hbm.at[SEQ - 1, :, pl.ds(b0, bw)], osem[1]
        ).wait()

    return k


def kernel(x, lut):
    B, seq = x.shape
    assert seq == SEQ and lut.shape[1] == D
    table2 = lut.reshape(lut.shape[0] // 2, 2 * D)
    out_t = _emb_kernel(B)(x.T.astype(jnp.int32), table2)
    return out_t.transpose(2, 0, 1)


# speedup vs baseline: 1.0720x; 1.0720x over previous
import functools

import jax
import jax.numpy as jnp
from jax import lax
from jax.experimental import pallas as pl
from jax.experimental.pallas import tpu as pltpu
from jax.experimental.pallas import tpu_sc as plsc

D = 64
SEQ = 200
SCALE = 8.0
LANES = 16
NG = 128 // LANES
DU = 4

_info = plsc.get_sparse_core_info()
_NC = _info.num_cores
_NS = _info.num_subcores
_NW = _NC * _NS


def _emb_kernel(B: int):
    bw = B // _NW
    mesh = plsc.VectorSubcoreMesh(core_axis_name="c", subcore_axis_name="s")

    @functools.partial(
        pl.kernel,
        mesh=mesh,
        out_type=jax.ShapeDtypeStruct((SEQ, D, B), jnp.float32),
        scratch_types=[
            pltpu.VMEM((bw,), jnp.int32),
            pltpu.VMEM((bw,), jnp.int32),
            pltpu.VMEM((bw,), jnp.int32),
            pltpu.VMEM((bw,), jnp.int32),
            pltpu.VMEM((bw, 2 * D), jnp.float32),
            pltpu.VMEM((bw, 2 * D), jnp.float32),
            pltpu.VMEM((D, bw), jnp.float32),
            pltpu.VMEM((D, bw), jnp.float32),
            pltpu.SemaphoreType.DMA,
            pltpu.SemaphoreType.DMA,
            pltpu.SemaphoreType.DMA,
            pltpu.SemaphoreType.DMA,
        ],
        compiler_params=pltpu.CompilerParams(
            use_tc_tiling_on_sc=False, needs_layout_passes=False
        ),
    )
    def k(idx_hbm, table_hbm, out_hbm,
          idxA, idxB, halfA, halfB, pairsA, pairsB, slabA, slabB,
          gsemA, gsemB, osemA, osemB):
        wid = lax.axis_index("s") * _NC + lax.axis_index("c")
        b0 = wid * bw

        def stage(s, idx_v, half_v, pairs_v, gsem):
            pltpu.sync_copy(idx_hbm.at[s, pl.ds(b0, bw)], idx_v)
            for g in range(NG):
                sl = pl.ds(g * LANES, LANES)
                half_v[sl] = lax.shift_right_logical(idx_v[sl], 1)
            return pltpu.async_copy(table_hbm.at[half_v], pairs_v, gsem)

        def transpose(idx_v, pairs_v, slab_v):
            for g in range(NG):
                sl = pl.ds(g * LANES, LANES)
                rowi = jax.lax.iota(jnp.int32, LANES) + (g * LANES)
                coli = (idx_v[sl] & 1) * D

                def dbody(i, _):
                    dbase = i * DU
                    for u in range(DU):
                        d = dbase + u
                        vals = plsc.load_gather(pairs_v, [rowi, coli + d])
                        slab_v[d, sl] = vals * SCALE
                    return 0

                lax.fori_loop(0, D // DU, dbody, 0)

        def pair_body(p, _):
            sA = 2 * p
            sB = 2 * p + 1
            cpA = stage(sA, idxA, halfA, pairsA, gsemA)
            cpB = stage(sB, idxB, halfB, pairsB, gsemB)

            cpA.wait()
            @pl.when(p >= 1)
            def _():
                pltpu.make_async_copy(
                    slabA, out_hbm.at[sA - 2, :, pl.ds(b0, bw)], osemA
                ).wait()

            transpose(idxA, pairsA, slabA)
            pltpu.make_async_copy(
                slabA, out_hbm.at[sA, :, pl.ds(b0, bw)], osemA
            ).start()

            cpB.wait()
            @pl.when(p >= 1)
            def _():
                pltpu.make_async_copy(
                    slabB, out_hbm.at[sB - 2, :, pl.ds(b0, bw)], osemB
                ).wait()

            transpose(idxB, pairsB, slabB)
            pltpu.make_async_copy(
                slabB, out_hbm.at[sB, :, pl.ds(b0, bw)], osemB
            ).start()
            return 0

        lax.fori_loop(0, SEQ // 2, pair_body, 0)

        pltpu.make_async_copy(
            slabA, out_hbm.at[SEQ - 2, :, pl.ds(b0, bw)], osemA
        ).wait()
        pltpu.make_async_copy(
            slabB, out_hbm.at[SEQ - 1, :, pl.ds(b0, bw)], osemB
        ).wait()

    return k


def kernel(x, lut):
    B, seq = x.shape
    assert seq == SEQ and lut.shape[1] == D
    table2 = lut.reshape(lut.shape[0] // 2, 2 * D)
    out_t = _emb_kernel(B)(x.T.astype(jnp.int32), table2)
    return out_t.transpose(2, 0, 1)


# speedup vs baseline: 1.9829x; 1.8498x over previous
import functools

import jax
import jax.numpy as jnp
from jax import lax
from jax.experimental import pallas as pl
from jax.experimental.pallas import tpu as pltpu
from jax.experimental.pallas import tpu_sc as plsc

D = 64
SEQ = 200
SCALE = 8.0
LANES = 16

_info = plsc.get_sparse_core_info()
_NC = _info.num_cores
_NS = _info.num_subcores
_NW = _NC * _NS

S = 4
G = 40
GPS = SEQ // G


def _emb_kernel(B: int, V: int):
    sent_per_w = B // _NW
    n_chunks = sent_per_w // S
    n_pairs = n_chunks // 2
    mesh = plsc.VectorSubcoreMesh(core_axis_name="c", subcore_axis_name="s")

    @functools.partial(
        pl.kernel,
        mesh=mesh,
        out_type=jax.ShapeDtypeStruct((B, SEQ, D), jnp.float32),
        scratch_types=[
            pltpu.VMEM((S, SEQ), jnp.int32),
            pltpu.VMEM((S, SEQ, D), jnp.float32),
            pltpu.VMEM((S, SEQ, D), jnp.float32),
            pltpu.SemaphoreType.DMA,
            pltpu.SemaphoreType.DMA,
            pltpu.SemaphoreType.DMA,
        ],
        compiler_params=pltpu.CompilerParams(use_tc_tiling_on_sc=False),
    )
    def k(idx_hbm, table_hbm, out_hbm, idx_v, rows0, rows1, gsem, osem0, osem1):
        wid = lax.axis_index("s") * _NC + lax.axis_index("c")
        base = wid * sent_per_w
        rows = (rows0, rows1)
        osem = (osem0, osem1)

        def chunk_step(c, b):
            s0 = base + c * S
            @pl.when(c >= 2)
            def _():
                pltpu.make_async_copy(
                    rows[b], out_hbm.at[pl.ds(s0 - 2 * S, S)], osem[b]
                ).wait()

            pltpu.sync_copy(idx_hbm.at[pl.ds(s0, S)], idx_v)
            copies = []
            for s in range(S):
                for h in range(GPS):
                    copies.append(pltpu.async_copy(
                        table_hbm.at[idx_v.at[s, pl.ds(h * G, G)]],
                        rows[b].at[s, pl.ds(h * G, G)],
                        gsem,
                    ))
            for cp in copies:
                cp.wait()

            for s in range(S):
                @plsc.parallel_loop(0, SEQ, unroll=4)
                def _(r):
                    for cc in range(D // LANES):
                        sl = pl.ds(cc * LANES, LANES)
                        rows[b][s, r, sl] = rows[b][s, r, sl] * SCALE

            pltpu.make_async_copy(
                rows[b], out_hbm.at[pl.ds(s0, S)], osem[b]
            ).start()

        def pair_body(p, _):
            chunk_step(2 * p, 0)
            chunk_step(2 * p + 1, 1)
            return 0

        lax.fori_loop(0, n_pairs, pair_body, 0)

        last0 = base + (n_chunks - 2) * S
        pltpu.make_async_copy(rows[0], out_hbm.at[pl.ds(last0, S)], osem[0]).wait()
        pltpu.make_async_copy(
            rows[1], out_hbm.at[pl.ds(last0 + S, S)], osem[1]
        ).wait()

    return k


def kernel(x, lut):
    B, seq = x.shape
    assert seq == SEQ and lut.shape[1] == D
    return _emb_kernel(B, lut.shape[0])(x.astype(jnp.int32), lut)
